# Initial kernel scaffold; baseline (speedup 1.0000x reference)
import jax
import jax.numpy as jnp
from jax.experimental import pallas as pl


def kernel(logits, target):
    raise NotImplementedError("write your pallas kernel here")


# speedup vs baseline: 16.8102x; 16.8102x over previous
import functools

import jax
import jax.numpy as jnp
from jax.experimental import pallas as pl
from jax.experimental.pallas import tpu as pltpu

_SMOOTH = 1e-05
_K_FRAC = 10.0 / 100.0
_INT_MIN = -(2**31)
_INT_MAX = 2**31 - 1


def _dice_body(l0_ref, l1_ref, tgt_ref, out_ref, *, nb):
    b = pl.program_id(0)

    x = l1_ref[...] - l0_ref[...]
    fg = tgt_ref[...] == 1

    ki = jax.lax.bitcast_convert_type(x, jnp.int32)
    key = jnp.where(ki < 0, ki ^ jnp.int32(0x7FFFFFFF), ki)
    mkey = jnp.where(fg, key, jnp.int32(_INT_MAX))

    n = jnp.sum(fg.astype(jnp.int32))
    k = jnp.maximum(
        jnp.int32(1),
        jnp.floor(n.astype(jnp.float32) * jnp.float32(_K_FRAC)).astype(jnp.int32),
    )

    def body(_, carry):
        lo, hi = carry
        mid = (lo >> 1) + (hi >> 1) + (lo & hi & jnp.int32(1))
        c = jnp.sum((mkey <= mid).astype(jnp.int32))
        pred = c >= k
        return (jnp.where(pred, lo, mid + 1), jnp.where(pred, mid, hi))

    _, thr = jax.lax.fori_loop(
        0, 32, body, (jnp.int32(_INT_MIN), jnp.int32(_INT_MAX)))

    p = jax.nn.sigmoid(x)
    ign = jnp.logical_and(fg, key > thr)
    s_all = jnp.sum(p)
    pf = jnp.where(fg, p, jnp.float32(0.0))
    s_fg = jnp.sum(pf)
    s_ign = jnp.sum(jnp.where(ign, p, jnp.float32(0.0)))
    c_ign = jnp.sum(ign.astype(jnp.int32))

    inter = s_fg - s_ign
    union = (s_all - s_ign) + (n - c_ign).astype(jnp.float32)
    dice = (2.0 * inter + jnp.float32(_SMOOTH)) / (union + jnp.float32(_SMOOTH))

    @pl.when(b == 0)
    def _init():
        out_ref[0] = jnp.float32(0.0)

    out_ref[0] += dice

    @pl.when(b == nb - 1)
    def _fini():
        out_ref[0] = jnp.float32(1.0) - out_ref[0] / jnp.float32(nb)


def kernel(logits, target):
    logits = logits.astype(jnp.float32)
    B, _, H, W = logits.shape
    l0 = logits[:, 0]
    l1 = logits[:, 1]
    tgt = target.astype(jnp.int32)

    out = pl.pallas_call(
        functools.partial(_dice_body, nb=B),
        grid=(B,),
        in_specs=[
            pl.BlockSpec((1, H, W), lambda b: (b, 0, 0)),
            pl.BlockSpec((1, H, W), lambda b: (b, 0, 0)),
            pl.BlockSpec((1, H, W), lambda b: (b, 0, 0)),
        ],
        out_specs=pl.BlockSpec(memory_space=pltpu.SMEM),
        out_shape=jax.ShapeDtypeStruct((1,), jnp.float32),
    )(l0, l1, tgt)
    return out[0]


# speedup vs baseline: 31.6585x; 1.8833x over previous
import jax
import jax.numpy as jnp
from jax.experimental import pallas as pl
from jax.experimental.pallas import tpu as pltpu

_SMOOTH = 1e-05
_K_FRAC = 10.0 / 100.0
_INT_MIN = -(2**31)
_INT_MAX = 2**31 - 1


def _key_of(x):
    ki = jax.lax.bitcast_convert_type(x, jnp.int32)
    return jnp.where(ki < 0, ki ^ jnp.int32(0x7FFFFFFF), ki)


def _dice_body(l0_ref, l1_ref, tgt_ref, out_ref, mkey_ref):
    B = l0_ref.shape[0]

    ks = []
    for s in range(B):
        x = l1_ref[s] - l0_ref[s]
        fg = tgt_ref[s] == 1
        mkey_ref[s] = jnp.where(fg, _key_of(x), jnp.int32(_INT_MAX))
        n = jnp.sum(fg.astype(jnp.int32))
        ks.append(jnp.maximum(
            jnp.int32(1),
            jnp.floor(n.astype(jnp.float32) * jnp.float32(_K_FRAC)).astype(jnp.int32),
        ))

    def body(_, carry):
        los, his = carry
        nlos, nhis = [], []
        for s in range(B):
            lo, hi = los[s], his[s]
            mid = (lo >> 1) + (hi >> 1) + (lo & hi & jnp.int32(1))
            c = jnp.sum((mkey_ref[s] <= mid).astype(jnp.int32))
            pred = c >= ks[s]
            nlos.append(jnp.where(pred, lo, mid + 1))
            nhis.append(jnp.where(pred, mid, hi))
        return (tuple(nlos), tuple(nhis))

    init = (
        tuple(jnp.int32(_INT_MIN) for _ in range(B)),
        tuple(jnp.int32(_INT_MAX) for _ in range(B)),
    )
    _, thrs = jax.lax.fori_loop(0, 32, body, init)

    acc = jnp.float32(0.0)
    for s in range(B):
        x = l1_ref[s] - l0_ref[s]
        fg = tgt_ref[s] == 1
        key = _key_of(x)
        p = jax.nn.sigmoid(x)
        ign = jnp.logical_and(fg, key > thrs[s])
        s_all = jnp.sum(p)
        s_fg = jnp.sum(jnp.where(fg, p, jnp.float32(0.0)))
        s_ign = jnp.sum(jnp.where(ign, p, jnp.float32(0.0)))
        n = jnp.sum(fg.astype(jnp.int32))
        c_ign = jnp.sum(ign.astype(jnp.int32))
        inter = s_fg - s_ign
        union = (s_all - s_ign) + (n - c_ign).astype(jnp.float32)
        acc += (2.0 * inter + jnp.float32(_SMOOTH)) / (union + jnp.float32(_SMOOTH))

    out_ref[0] = jnp.float32(1.0) - acc / jnp.float32(B)


def kernel(logits, target):
    logits = logits.astype(jnp.float32)
    B, _, H, W = logits.shape
    l0 = logits[:, 0].reshape(B, (H * W) // 512, 512)
    l1 = logits[:, 1].reshape(B, (H * W) // 512, 512)
    tgt = target.astype(jnp.int32).reshape(B, (H * W) // 512, 512)

    out = pl.pallas_call(
        _dice_body,
        in_specs=[
            pl.BlockSpec(l0.shape, lambda: (0, 0, 0)),
            pl.BlockSpec(l1.shape, lambda: (0, 0, 0)),
            pl.BlockSpec(tgt.shape, lambda: (0, 0, 0)),
        ],
        out_specs=pl.BlockSpec(memory_space=pltpu.SMEM),
        out_shape=jax.ShapeDtypeStruct((1,), jnp.float32),
        scratch_shapes=[pltpu.VMEM((B, (H * W) // 512, 512), jnp.int32)],
    )(l0, l1, tgt)
    return out[0]


# speedup vs baseline: 38.0133x; 1.2007x over previous
import jax
import jax.numpy as jnp
from jax.experimental import pallas as pl
from jax.experimental.pallas import tpu as pltpu

_SMOOTH = 1e-05
_K_FRAC = 10.0 / 100.0
_INT_MAX = 2**31 - 1


def _key_of(x):
    ki = jax.lax.bitcast_convert_type(x, jnp.int32)
    return jnp.where(ki < 0, ki ^ jnp.int32(0x7FFFFFFF), ki)


def _count_le(plane_i16, mid_i32):
    t = (plane_i16 <= mid_i32.astype(jnp.int16)).astype(jnp.int16)
    while t.shape[0] > 16:
        h = t.shape[0] // 2
        t = t[:h] + t[h:]
    return jnp.sum(t.astype(jnp.int32))


def _search16(plane_refs, ks, B, lo0, hi0):
    def body(_, carry):
        los, his, cbls = carry
        nlo, nhi, ncb = [], [], []
        for s in range(B):
            lo, hi, cbl = los[s], his[s], cbls[s]
            mid = (lo + hi) >> 1
            c = _count_le(plane_refs(s), mid)
            pred = c >= ks[s]
            nlo.append(jnp.where(pred, lo, mid + 1))
            nhi.append(jnp.where(pred, mid, hi))
            ncb.append(jnp.where(pred, cbl, c))
        return (tuple(nlo), tuple(nhi), tuple(ncb))

    init = (
        tuple(jnp.int32(lo0) for _ in range(B)),
        tuple(jnp.int32(hi0) for _ in range(B)),
        tuple(jnp.int32(0) for _ in range(B)),
    )
    _, thrs, cbls = jax.lax.fori_loop(0, 16, body, init)
    return thrs, cbls


def _dice_body(l0_ref, l1_ref, tgt_ref, out_ref, mkey_ref, h16_ref):
    B = l0_ref.shape[0]

    ks = []
    for s in range(B):
        x = l1_ref[s] - l0_ref[s]
        fg = tgt_ref[s] == 1
        mkey = jnp.where(fg, _key_of(x), jnp.int32(_INT_MAX))
        mkey_ref[s] = mkey
        h16_ref[s] = (mkey >> 16).astype(jnp.int16)
        n = jnp.sum(fg.astype(jnp.int32))
        ks.append(jnp.maximum(
            jnp.int32(1),
            jnp.floor(n.astype(jnp.float32) * jnp.float32(_K_FRAC)).astype(jnp.int32),
        ))

    buckets, cbls = _search16(lambda s: h16_ref[s], ks, B, -(2**15), 2**15 - 1)

    for s in range(B):
        mkey = mkey_ref[s]
        hk = mkey >> 16
        low = (mkey & jnp.int32(0xFFFF)) - jnp.int32(32768)
        key2 = jnp.where(
            hk == buckets[s], low,
            jnp.where(hk < buckets[s], jnp.int32(-32768), jnp.int32(32767)),
        )
        h16_ref[s] = key2.astype(jnp.int16)

    lows, _ = _search16(lambda s: h16_ref[s], ks, B, -(2**15), 2**15 - 1)

    acc = jnp.float32(0.0)
    for s in range(B):
        thr = (buckets[s] << 16) | (lows[s] + jnp.int32(32768))
        x = l1_ref[s] - l0_ref[s]
        fg = tgt_ref[s] == 1
        key = _key_of(x)
        p = jax.nn.sigmoid(x)
        ign = jnp.logical_and(fg, key > thr)
        s_all = jnp.sum(p)
        s_fg = jnp.sum(jnp.where(fg, p, jnp.float32(0.0)))
        s_ign = jnp.sum(jnp.where(ign, p, jnp.float32(0.0)))
        n = jnp.sum(fg.astype(jnp.int32))
        c_ign = jnp.sum(ign.astype(jnp.int32))
        inter = s_fg - s_ign
        union = (s_all - s_ign) + (n - c_ign).astype(jnp.float32)
        acc += (2.0 * inter + jnp.float32(_SMOOTH)) / (union + jnp.float32(_SMOOTH))

    out_ref[0] = jnp.float32(1.0) - acc / jnp.float32(B)


def kernel(logits, target):
    logits = logits.astype(jnp.float32)
    B, _, H, W = logits.shape
    R = (H * W) // 512
    l0 = logits[:, 0].reshape(B, R, 512)
    l1 = logits[:, 1].reshape(B, R, 512)
    tgt = target.astype(jnp.int32).reshape(B, R, 512)

    out = pl.pallas_call(
        _dice_body,
        in_specs=[
            pl.BlockSpec(l0.shape, lambda: (0, 0, 0)),
            pl.BlockSpec(l1.shape, lambda: (0, 0, 0)),
            pl.BlockSpec(tgt.shape, lambda: (0, 0, 0)),
        ],
        out_specs=pl.BlockSpec(memory_space=pltpu.SMEM),
        out_shape=jax.ShapeDtypeStruct((1,), jnp.float32),
        scratch_shapes=[
            pltpu.VMEM((B, R, 512), jnp.int32),
            pltpu.VMEM((B, R, 512), jnp.int16),
        ],
    )(l0, l1, tgt)
    return out[0]
